# baseline (speedup 1.0000x reference)
import functools

import jax
import jax.numpy as jnp
from jax import lax
from jax.experimental import pallas as pl
from jax.experimental.pallas import tpu as pltpu
from jax.experimental.pallas import tpu_sc as plsc

RATE = 0.1
SCALE = 1.0 / (1.0 - RATE)

B = 4096
L = 50
DIM = 32
N = B * L
NW = 32
PER_W = N // NW
IDXW = 128
CHUNK = 640
NSTREAM = CHUNK // IDXW
NCHUNK = PER_W // CHUNK
IDS_ROWS_PER_CHUNK = CHUNK // IDXW


def _sc_body(ids_hbm, mask_hbm, w_hbm, out_hbm, idx_v, rows_v, mask_v, sem):
    c = lax.axis_index("c")
    s = lax.axis_index("s")
    wid = s * 2 + c
    flat_base = wid * PER_W

    def chunk_body(k, carry):
        pltpu.sync_copy(ids_hbm.at[pl.ds(flat_base + k * CHUNK, CHUNK)], idx_v)
        copies = [
            pltpu.async_copy(
                w_hbm.at[idx_v.at[pl.ds(j * IDXW, IDXW)]],
                rows_v.at[pl.ds(j * IDXW, IDXW)],
                sem,
            )
            for j in range(NSTREAM)
        ]
        off = flat_base + k * CHUNK
        pltpu.sync_copy(mask_hbm.at[pl.ds(off, CHUNK)], mask_v)
        for cp in copies:
            cp.wait()

        def row_body(r, inner):
            for h in range(DIM // 16):
                sl = pl.ds(h * 16, 16)
                u = mask_v[r, sl]
                scale = jnp.where(u >= RATE, SCALE, 0.0)
                rows_v[r, sl] = rows_v[r, sl] * scale
            return inner

        lax.fori_loop(0, CHUNK, row_body, 0, unroll=4)

        pltpu.sync_copy(rows_v, out_hbm.at[pl.ds(off, CHUNK)])
        return carry

    lax.fori_loop(0, NCHUNK, chunk_body, 0)


@functools.partial(jax.jit, static_argnames=())
def _run(ids2d, mask2d, w):
    kern = functools.partial(
        pl.kernel,
        mesh=plsc.VectorSubcoreMesh(core_axis_name="c", subcore_axis_name="s"),
        out_type=jax.ShapeDtypeStruct((N, DIM), jnp.float32),
        compiler_params=pltpu.CompilerParams(use_tc_tiling_on_sc=False),
        scratch_types=[
            pltpu.VMEM((CHUNK,), jnp.int32),
            pltpu.VMEM((CHUNK, DIM), jnp.float32),
            pltpu.VMEM((CHUNK, DIM), jnp.float32),
            pltpu.SemaphoreType.DMA,
        ],
    )(_sc_body)
    return kern(ids2d, mask2d, w)


def kernel(ids, w, mask_u):
    ids2d = ids.reshape(N)
    mask2d = mask_u.reshape(N, DIM)
    out = _run(ids2d, mask2d, w)
    return out.reshape(B, L, DIM)


# speedup vs baseline: 1.3870x; 1.3870x over previous
import functools

import jax
import jax.numpy as jnp
from jax import lax
from jax.experimental import pallas as pl
from jax.experimental.pallas import tpu as pltpu
from jax.experimental.pallas import tpu_sc as plsc

RATE = 0.1
SCALE = 1.0 / (1.0 - RATE)

B = 4096
L = 50
DIM = 32
N = B * L
NW = 32
PER_W = N // NW
IDXW = 128
CHUNK = 640
NSTREAM = CHUNK // IDXW
NCHUNK = PER_W // CHUNK


def _sc_body(ids_hbm, mask_hbm, w_hbm, out_hbm, idx_v, rows_v, mask_v, out_v, sem):
    c = lax.axis_index("c")
    s = lax.axis_index("s")
    wid = s * 2 + c
    flat_base = wid * PER_W

    def chunk_body(k, carry):
        pltpu.sync_copy(ids_hbm.at[pl.ds(flat_base + k * CHUNK, CHUNK)], idx_v)
        copies = [
            pltpu.async_copy(
                w_hbm.at[idx_v.at[pl.ds(j * IDXW, IDXW)]],
                rows_v.at[pl.ds(j * IDXW, IDXW)],
                sem,
            )
            for j in range(NSTREAM)
        ]
        off = (flat_base + k * CHUNK) * DIM
        pltpu.sync_copy(mask_hbm.at[pl.ds(off, CHUNK * DIM)], mask_v)
        for cp in copies:
            cp.wait()

        def row_body(r, inner):
            for h in range(DIM // 16):
                f = r * DIM + h * 16
                u = mask_v[pl.ds(f, 16)]
                scale = jnp.where(u >= RATE, SCALE, 0.0)
                out_v[pl.ds(f, 16)] = rows_v[r, pl.ds(h * 16, 16)] * scale
            return inner

        lax.fori_loop(0, CHUNK, row_body, 0, unroll=4)

        pltpu.sync_copy(out_v, out_hbm.at[pl.ds(off, CHUNK * DIM)])
        return carry

    lax.fori_loop(0, NCHUNK, chunk_body, 0)


@jax.jit
def _run(ids1d, mask1d, w):
    kern = functools.partial(
        pl.kernel,
        mesh=plsc.VectorSubcoreMesh(core_axis_name="c", subcore_axis_name="s"),
        out_type=jax.ShapeDtypeStruct((N * DIM,), jnp.float32),
        compiler_params=pltpu.CompilerParams(use_tc_tiling_on_sc=False),
        scratch_types=[
            pltpu.VMEM((CHUNK,), jnp.int32),
            pltpu.VMEM((CHUNK, DIM), jnp.float32),
            pltpu.VMEM((CHUNK * DIM,), jnp.float32),
            pltpu.VMEM((CHUNK * DIM,), jnp.float32),
            pltpu.SemaphoreType.DMA,
        ],
    )(_sc_body)
    return kern(ids1d, mask1d, w)


def kernel(ids, w, mask_u):
    ids1d = ids.reshape(N)
    mask1d = mask_u.reshape(N * DIM)
    out = _run(ids1d, mask1d, w)
    return out.reshape(B, L, DIM)


# speedup vs baseline: 1.3929x; 1.0042x over previous
import functools

import jax
import jax.numpy as jnp
from jax import lax
from jax.experimental import pallas as pl
from jax.experimental.pallas import tpu as pltpu
from jax.experimental.pallas import tpu_sc as plsc

RATE = 0.1
SCALE = 1.0 / (1.0 - RATE)

B = 4096
L = 50
DIM = 32
NW = 32
BPW = B // NW
BCHUNK = 16
NCHUNK = BPW // BCHUNK


def _sc_body(ids_hbm, mask_hbm, w_hbm, out_hbm, idx_v, rows_v, mask_v, out_v, sem):
    c = lax.axis_index("c")
    s = lax.axis_index("s")
    wid = s * 2 + c
    b_base = wid * BPW

    def chunk_body(k, carry):
        b0 = b_base + k * BCHUNK
        pltpu.sync_copy(ids_hbm.at[pl.ds(b0, BCHUNK)], idx_v)
        copies = [
            pltpu.async_copy(w_hbm.at[idx_v.at[j]], rows_v.at[j], sem)
            for j in range(BCHUNK)
        ]
        pltpu.sync_copy(mask_hbm.at[pl.ds(b0, BCHUNK)], mask_v)
        for cp in copies:
            cp.wait()

        def row_body(bi, inner):
            def l_body(l, inner2):
                for h in range(DIM // 16):
                    sl = pl.ds(h * 16, 16)
                    u = mask_v[bi, l, sl]
                    scale = jnp.where(u >= RATE, SCALE, 0.0)
                    out_v[bi, l, sl] = rows_v[bi, l, sl] * scale
                return inner2

            return lax.fori_loop(0, L, l_body, inner, unroll=5)

        lax.fori_loop(0, BCHUNK, row_body, 0)

        pltpu.sync_copy(out_v, out_hbm.at[pl.ds(b0, BCHUNK)])
        return carry

    lax.fori_loop(0, NCHUNK, chunk_body, 0)


@jax.jit
def _run(ids, mask_u, w):
    kern = functools.partial(
        pl.kernel,
        mesh=plsc.VectorSubcoreMesh(core_axis_name="c", subcore_axis_name="s"),
        out_type=jax.ShapeDtypeStruct((B, L, DIM), jnp.float32),
        compiler_params=pltpu.CompilerParams(use_tc_tiling_on_sc=False),
        scratch_types=[
            pltpu.VMEM((BCHUNK, L), jnp.int32),
            pltpu.VMEM((BCHUNK, L, DIM), jnp.float32),
            pltpu.VMEM((BCHUNK, L, DIM), jnp.float32),
            pltpu.VMEM((BCHUNK, L, DIM), jnp.float32),
            pltpu.SemaphoreType.DMA,
        ],
    )(_sc_body)
    return kern(ids, mask_u, w)


def kernel(ids, w, mask_u):
    return _run(ids, mask_u, w)
